# Initial kernel scaffold; baseline (speedup 1.0000x reference)
import jax
import jax.numpy as jnp
from jax.experimental import pallas as pl


def kernel(coords, features, batch_ids, times, tau, ln_g, ln_b, w_sp1, b_sp1, w_sp2, b_sp2, w_m1, b_m1, w_m2, b_m2, w_m3, b_m3, w_m4, b_m4, w_i1, b_i1, w_i2, b_i2, w_n1, b_n1, w_n2, b_n2):
    raise NotImplementedError("write your pallas kernel here")


# speedup vs baseline: 1.0260x; 1.0260x over previous
import functools

import jax
import jax.numpy as jnp
from jax.experimental import pallas as pl
from jax.experimental.pallas import tpu as pltpu

N_POINTS = 32768
BLK = 512


def _mlp_body(xyzt_ref, feats_ref, bids_ref,
              lng_ref, lnb_ref,
              wsp1_ref, bsp1_ref, wsp2_ref, bsp2_ref,
              wm1_ref, bm1_ref, wm2_ref, bm2_ref,
              wm3_ref, bm3_ref, wm4_ref, bm4_ref,
              wi1_ref, bi1_ref, wi2_ref, bi2_ref,
              pf_out_ref, sc_out_ref, cnt_out_ref):
    i = pl.program_id(0)
    xyzt = xyzt_ref[...]
    mu = jnp.mean(xyzt, axis=1, keepdims=True)
    var = jnp.mean((xyzt - mu) ** 2, axis=1, keepdims=True)
    xn = (xyzt - mu) / jnp.sqrt(var + 1e-5)
    xn = xn * lng_ref[...] + lnb_ref[...]
    f32 = jnp.float32
    h = jax.nn.relu(jnp.dot(xn, wsp1_ref[...], preferred_element_type=f32)
                    + bsp1_ref[...])
    sp = jax.nn.relu(jnp.dot(h, wsp2_ref[...], preferred_element_type=f32)
                     + bsp2_ref[...])
    pf = jnp.concatenate([feats_ref[...], sp], axis=1)
    pf = jax.nn.relu(jnp.dot(pf, wm1_ref[...], preferred_element_type=f32)
                     + bm1_ref[...])
    pf = jax.nn.relu(jnp.dot(pf, wm2_ref[...], preferred_element_type=f32)
                     + bm2_ref[...])
    pf = jax.nn.relu(jnp.dot(pf, wm3_ref[...], preferred_element_type=f32)
                     + bm3_ref[...])
    pf = jnp.dot(pf, wm4_ref[...], preferred_element_type=f32) + bm4_ref[...]
    pf_out_ref[...] = pf
    hi = jax.nn.relu(jnp.dot(pf, wi1_ref[...], preferred_element_type=f32)
                     + bi1_ref[...])
    sc_out_ref[...] = jnp.dot(hi, wi2_ref[...], preferred_element_type=f32) \
        + bi2_ref[...]
    bids = bids_ref[...]
    bins = jax.lax.broadcasted_iota(jnp.int32, (1, 8), 1)
    part = jnp.sum((bids == bins).astype(jnp.float32), axis=0,
                   keepdims=True)
    prev = jnp.where(i == 0, jnp.zeros_like(part), cnt_out_ref[...])
    cnt_out_ref[...] = prev + part


def _run_mlp(xyzt, features, batch_ids,
             ln_g, ln_b, w_sp1, b_sp1, w_sp2, b_sp2,
             w_m1, b_m1, w_m2, b_m2, w_m3, b_m3, w_m4, b_m4,
             w_i1, b_i1, w_i2, b_i2):
    n = xyzt.shape[0]
    grid = n // BLK
    row = lambda i: (i, 0)
    fix = lambda i: (0, 0)

    def wspec(shape):
        return pl.BlockSpec(shape, fix)

    ws = [ln_g.reshape(1, 4), ln_b.reshape(1, 4),
          w_sp1.T, b_sp1.reshape(1, -1), w_sp2.T, b_sp2.reshape(1, -1),
          w_m1.T, b_m1.reshape(1, -1), w_m2.T, b_m2.reshape(1, -1),
          w_m3.T, b_m3.reshape(1, -1), w_m4.T, b_m4.reshape(1, -1),
          w_i1.T, b_i1.reshape(1, -1), w_i2.T, b_i2.reshape(1, -1)]
    in_specs = [pl.BlockSpec((BLK, 4), row),
                pl.BlockSpec((BLK, 256), row),
                pl.BlockSpec((BLK, 1), row)] + [wspec(w.shape) for w in ws]
    out_specs = [pl.BlockSpec((BLK, 768), row),
                 pl.BlockSpec((BLK, 1), row),
                 pl.BlockSpec((1, 8), fix)]
    out_shape = [jax.ShapeDtypeStruct((n, 768), jnp.float32),
                 jax.ShapeDtypeStruct((n, 1), jnp.float32),
                 jax.ShapeDtypeStruct((1, 8), jnp.float32)]
    pf, sc, cnt = pl.pallas_call(
        _mlp_body, grid=(grid,),
        in_specs=in_specs, out_specs=out_specs, out_shape=out_shape,
    )(xyzt, features, batch_ids.reshape(n, 1), *ws)
    return pf, sc[:, 0], cnt[0].astype(jnp.int32)


def kernel(coords, features, batch_ids, times, tau, ln_g, ln_b,
           w_sp1, b_sp1, w_sp2, b_sp2,
           w_m1, b_m1, w_m2, b_m2, w_m3, b_m3, w_m4, b_m4,
           w_i1, b_i1, w_i2, b_i2, w_n1, b_n1, w_n2, b_n2):
    n = coords.shape[0]
    bsz, k_eff, kk = 8, 128, 16
    xyzt = jnp.concatenate([coords, times], axis=-1)
    point_feats, scores, counts = _run_mlp(
        xyzt, features, batch_ids, ln_g, ln_b,
        w_sp1, b_sp1, w_sp2, b_sp2, w_m1, b_m1, w_m2, b_m2,
        w_m3, b_m3, w_m4, b_m4, w_i1, b_i1, w_i2, b_i2)

    offsets = jnp.concatenate(
        [jnp.zeros((1,), counts.dtype), jnp.cumsum(counts)[:-1]])
    neg_inf = jnp.finfo(jnp.float32).min
    pos = jnp.arange(n, dtype=jnp.int32)
    pos_in_b = pos - offsets[batch_ids]
    S = jnp.full((bsz, n), neg_inf, jnp.float32).at[batch_ids, pos_in_b].set(scores)
    top_vals, top_pos = jax.lax.top_k(S, k_eff)
    sel_idx = jnp.where(top_pos < counts[:, None],
                        offsets[:, None] + top_pos, n).astype(jnp.int32)
    sel_mask = top_pos < counts[:, None]
    xyzt_pad = jnp.concatenate([xyzt, jnp.zeros((1, 4), xyzt.dtype)], axis=0)
    sel_xyzt = xyzt_pad[sel_idx]
    xyzt_table = jnp.zeros((bsz, n, 4), xyzt.dtype).at[batch_ids, pos_in_b].set(xyzt)
    valid = jnp.arange(n)[None, :] < counts[:, None]
    d = jnp.sum((sel_xyzt[:, :, None, :3] - xyzt_table[:, None, :, :3]) ** 2, axis=-1)
    d = jnp.where(valid[:, None, :], d, jnp.inf)
    _, nb_pos = jax.lax.top_k(-d, kk)
    nb_valid = jnp.take_along_axis(
        jnp.broadcast_to(valid[:, None, :], (bsz, k_eff, n)), nb_pos, axis=2)
    nb_global = jnp.where(nb_valid, offsets[:, None, None] + nb_pos, n)
    feats_pad = jnp.concatenate(
        [point_feats, jnp.zeros((1, 768), point_feats.dtype)], axis=0)
    nb_feats = feats_pad[nb_global]
    nbm = nb_valid[..., None].astype(jnp.float32)
    agg = jnp.sum(nb_feats * nbm, axis=2) / jnp.clip(jnp.sum(nbm, axis=2), 1.0, None)
    neigh = jax.nn.relu(agg @ w_n1.T + b_n1) @ w_n2.T + b_n2
    sel_feats = feats_pad[sel_idx]
    tokens = (sel_feats + neigh) * sel_mask[..., None].astype(jnp.float32)
    centroids = sel_xyzt * sel_mask[..., None].astype(sel_xyzt.dtype)
    return tokens, centroids, sel_mask


# speedup vs baseline: 18.2458x; 17.7841x over previous
import functools

import jax
import jax.numpy as jnp
from jax import lax
from jax.experimental import pallas as pl
from jax.experimental.pallas import tpu as pltpu
from jax.experimental.pallas import tpu_sc as plsc

N_POINTS = 32768
BLK = 512
W_WIN = 4096
NW = 8
NPAD = N_POINTS + W_WIN + 8
NSC = 32
F_INF = float("inf")


def _mlp_body(xyzt_ref, feats_ref, bids_ref,
              lng_ref, lnb_ref,
              wsp1_ref, bsp1_ref, wsp2_ref, bsp2_ref,
              wm1_ref, bm1_ref, wm2_ref, bm2_ref,
              wm3_ref, bm3_ref, wm4_ref, bm4_ref,
              wi1_ref, bi1_ref, wi2_ref, bi2_ref,
              pf_out_ref, sc_out_ref, cnt_out_ref):
    i = pl.program_id(0)
    xyzt = xyzt_ref[...]
    mu = jnp.mean(xyzt, axis=1, keepdims=True)
    var = jnp.mean((xyzt - mu) ** 2, axis=1, keepdims=True)
    xn = (xyzt - mu) / jnp.sqrt(var + 1e-5)
    xn = xn * lng_ref[...] + lnb_ref[...]
    f32 = jnp.float32
    h = jax.nn.relu(jnp.dot(xn, wsp1_ref[...], preferred_element_type=f32)
                    + bsp1_ref[...])
    sp = jax.nn.relu(jnp.dot(h, wsp2_ref[...], preferred_element_type=f32)
                     + bsp2_ref[...])
    pf = jnp.concatenate([feats_ref[...], sp], axis=1)
    pf = jax.nn.relu(jnp.dot(pf, wm1_ref[...], preferred_element_type=f32)
                     + bm1_ref[...])
    pf = jax.nn.relu(jnp.dot(pf, wm2_ref[...], preferred_element_type=f32)
                     + bm2_ref[...])
    pf = jax.nn.relu(jnp.dot(pf, wm3_ref[...], preferred_element_type=f32)
                     + bm3_ref[...])
    pf = jnp.dot(pf, wm4_ref[...], preferred_element_type=f32) + bm4_ref[...]
    pf_out_ref[...] = pf
    hi = jax.nn.relu(jnp.dot(pf, wi1_ref[...], preferred_element_type=f32)
                     + bi1_ref[...])
    sc_out_ref[...] = jnp.dot(hi, wi2_ref[...], preferred_element_type=f32) \
        + bi2_ref[...]
    bids = bids_ref[...]
    bins = jax.lax.broadcasted_iota(jnp.int32, (1, 8), 1)
    part = jnp.sum((bids == bins).astype(jnp.float32), axis=0, keepdims=True)
    prev = jnp.where(i == 0, jnp.zeros_like(part), cnt_out_ref[...])
    cnt_out_ref[...] = prev + part


def _run_mlp(xyzt, features, batch_ids,
             ln_g, ln_b, w_sp1, b_sp1, w_sp2, b_sp2,
             w_m1, b_m1, w_m2, b_m2, w_m3, b_m3, w_m4, b_m4,
             w_i1, b_i1, w_i2, b_i2):
    n = xyzt.shape[0]
    grid = n // BLK
    row = lambda i: (i, 0)
    fix = lambda i: (0, 0)

    def wspec(shape):
        return pl.BlockSpec(shape, fix)

    ws = [ln_g.reshape(1, 4), ln_b.reshape(1, 4),
          w_sp1.T, b_sp1.reshape(1, -1), w_sp2.T, b_sp2.reshape(1, -1),
          w_m1.T, b_m1.reshape(1, -1), w_m2.T, b_m2.reshape(1, -1),
          w_m3.T, b_m3.reshape(1, -1), w_m4.T, b_m4.reshape(1, -1),
          w_i1.T, b_i1.reshape(1, -1), w_i2.T, b_i2.reshape(1, -1)]
    in_specs = [pl.BlockSpec((BLK, 4), row),
                pl.BlockSpec((BLK, 256), row),
                pl.BlockSpec((BLK, 1), row)] + [wspec(w.shape) for w in ws]
    out_specs = [pl.BlockSpec((BLK, 768), row),
                 pl.BlockSpec((BLK, 1), row),
                 pl.BlockSpec((1, 8), fix)]
    out_shape = [jax.ShapeDtypeStruct((n, 768), jnp.float32),
                 jax.ShapeDtypeStruct((n, 1), jnp.float32),
                 jax.ShapeDtypeStruct((1, 8), jnp.float32)]
    pf, sc, cnt = pl.pallas_call(
        _mlp_body, grid=(grid,),
        in_specs=in_specs, out_specs=out_specs, out_shape=out_shape,
    )(xyzt, features, batch_ids.reshape(n, 1), *ws)
    return pf, sc[:, 0], cnt[0].astype(jnp.int32)


def _axc():
    return lax.axis_index("c")


def _axs():
    return lax.axis_index("s")


def _sload(ref, idx):
    return ref[pl.ds(idx, 16)][0]


def _knn_body(x_hbm, y_hbm, z_hbm, xyzt128_hbm, feats_hbm, sel_hbm, co_hbm,
              agg_hbm, selfeats_hbm, cent_hbm,
              xbuf, ybuf, zbuf, cobuf, selbuf, gidx, cidx, rows_v,
              centrows, aggrow, zrow, sem):
    wid = _axc() * 16 + _axs()
    seg = wid // 4
    q = wid % 4
    iota = jnp.arange(16, dtype=jnp.int32)

    pltpu.sync_copy(co_hbm, cobuf.at[pl.ds(0, 128)])
    cnt = _sload(cobuf, seg)
    off = _sload(cobuf, 8 + seg)
    sh = off % 8
    off_al = pl.multiple_of(off - sh, 8)
    lim = sh + cnt

    zf = jnp.zeros((16,), jnp.float32)

    def _zero(i, _):
        zrow[pl.ds(i * 16, 16)] = zf
        aggrow[pl.ds(i * 16, 16)] = zf
        return 0

    lax.fori_loop(0, 48, _zero, 0)

    for w in range(NW):

        @pl.when(w * W_WIN < lim)
        def _copy():
            wb = pl.multiple_of(off_al + w * W_WIN, 8)
            pltpu.sync_copy(x_hbm.at[pl.ds(wb, W_WIN)],
                            xbuf.at[pl.ds(w * W_WIN, W_WIN)])
            pltpu.sync_copy(y_hbm.at[pl.ds(wb, W_WIN)],
                            ybuf.at[pl.ds(w * W_WIN, W_WIN)])
            pltpu.sync_copy(z_hbm.at[pl.ds(wb, W_WIN)],
                            zbuf.at[pl.ds(w * W_WIN, W_WIN)])

    pltpu.sync_copy(sel_hbm, selbuf.at[pl.ds(0, 1024)])

    for h in range(2):
        selv = selbuf[pl.ds(wid * 32 + h * 16, 16)]
        cidx[...] = selv + off
        pltpu.async_copy(xyzt128_hbm.at[cidx], centrows, sem).wait()
        pltpu.sync_copy(centrows,
                        cent_hbm.at[pl.ds(wid * 32 + h * 16, 16), :])

    jlim = (lim + 15) // 16

    def _token(tloc, _):
        r = q * 32 + tloc
        gout = seg * 128 + r
        valid = r < cnt

        @pl.when(valid)
        def _do():
            pos = _sload(selbuf, wid * 32 + tloc)
            own_local = pos + sh
            own_g = off + pos
            tx = _sload(xbuf, own_local)
            ty = _sload(ybuf, own_local)
            tz = _sload(zbuf, own_local)

            def _scan(j, carry):
                bk, bv, th = carry
                base = j * 16
                xv = xbuf[pl.ds(base, 16)]
                yv = ybuf[pl.ds(base, 16)]
                zv = zbuf[pl.ds(base, 16)]
                loc = base + iota
                dx = xv - tx
                dy = yv - ty
                dz = zv - tz
                d = dx * dx + dy * dy + dz * dz
                ok = (loc >= sh) & (loc < lim)
                dm = jnp.where(ok, d, F_INF)
                hit = dm < th

                def _merge(args):
                    bk, bv, th = args
                    ck, cv = plsc.sort_key_val(dm, loc)
                    rk = jnp.flip(ck)
                    rv = jnp.flip(cv)
                    m = bk <= rk
                    nk = jnp.where(m, bk, rk)
                    nv = jnp.where(m, bv, rv)
                    sk, sv = plsc.sort_key_val(nk, nv)
                    nth = jnp.full((16,), jnp.max(sk))
                    return sk, sv, nth

                return lax.cond(jnp.any(hit), _merge, lambda a: a,
                                (bk, bv, th))

            bk0 = jnp.full((16,), F_INF, jnp.float32)
            bv0 = jnp.zeros((16,), jnp.int32)
            bk, bv, _th = lax.fori_loop(0, jlim, _scan, (bk0, bv0, bk0))

            nvalid = jnp.minimum(cnt, 16)
            gidx[...] = bv + (off - sh)
            pltpu.async_copy(feats_hbm.at[gidx], rows_v, sem).wait()

            def _zrow(j, _):
                for c in range(48):
                    rows_v[j, pl.ds(c * 16, 16)] = zf
                return 0

            lax.fori_loop(nvalid, 16, _zrow, 0)

            nf = nvalid.astype(jnp.float32)
            for c in range(48):
                acc = rows_v[0, pl.ds(c * 16, 16)]
                for j in range(1, 16):
                    acc = acc + rows_v[j, pl.ds(c * 16, 16)]
                aggrow[pl.ds(c * 16, 16)] = acc / nf
            pltpu.sync_copy(aggrow, agg_hbm.at[gout])
            pltpu.sync_copy(feats_hbm.at[own_g], selfeats_hbm.at[gout])

        @pl.when(jnp.logical_not(valid))
        def _skip():
            pltpu.sync_copy(zrow, agg_hbm.at[gout])
            pltpu.sync_copy(zrow, selfeats_hbm.at[gout])

        return 0

    lax.fori_loop(0, 32, _token, 0)


def _run_knn(xp, yp, zp, xyzt128, point_feats, selpos, counts, offsets):
    mesh = plsc.VectorSubcoreMesh(core_axis_name="c", subcore_axis_name="s",
                                  num_cores=2, num_subcores=16)
    co = jnp.concatenate(
        [counts, offsets, jnp.zeros((112,), jnp.int32)]).astype(jnp.int32)
    out_type = [jax.ShapeDtypeStruct((1024, 768), jnp.float32),
                jax.ShapeDtypeStruct((1024, 768), jnp.float32),
                jax.ShapeDtypeStruct((1024, 128), jnp.float32)]
    scratch = [pltpu.VMEM((NW * W_WIN + 16,), jnp.float32),
               pltpu.VMEM((NW * W_WIN + 16,), jnp.float32),
               pltpu.VMEM((NW * W_WIN + 16,), jnp.float32),
               pltpu.VMEM((144,), jnp.int32),
               pltpu.VMEM((1040,), jnp.int32),
               pltpu.VMEM((16,), jnp.int32),
               pltpu.VMEM((16,), jnp.int32),
               pltpu.VMEM((16, 768), jnp.float32),
               pltpu.VMEM((16, 128), jnp.float32),
               pltpu.VMEM((768,), jnp.float32),
               pltpu.VMEM((768,), jnp.float32),
               pltpu.SemaphoreType.DMA]
    fn = pl.kernel(_knn_body, out_type=out_type, mesh=mesh,
                   scratch_types=scratch,
                   compiler_params=pltpu.CompilerParams(
                       needs_layout_passes=False))
    return fn(xp, yp, zp, xyzt128, point_feats, selpos, co)


def _tok_body(agg_ref, sf_ref, m_ref, w1_ref, b1_ref, w2_ref, b2_ref,
              out_ref):
    f32 = jnp.float32
    h = jax.nn.relu(jnp.dot(agg_ref[...], w1_ref[...],
                            preferred_element_type=f32) + b1_ref[...])
    neigh = jnp.dot(h, w2_ref[...], preferred_element_type=f32) + b2_ref[...]
    out_ref[...] = (sf_ref[...] + neigh) * m_ref[...]


def _run_tok(agg, sel_feats, maskf, w_n1, b_n1, w_n2, b_n2):
    row = lambda i: (i, 0)
    fix = lambda i: (0, 0)
    in_specs = [pl.BlockSpec((512, 768), row),
                pl.BlockSpec((512, 768), row),
                pl.BlockSpec((512, 1), row),
                pl.BlockSpec((768, 768), fix),
                pl.BlockSpec((1, 768), fix),
                pl.BlockSpec((768, 768), fix),
                pl.BlockSpec((1, 768), fix)]
    return pl.pallas_call(
        _tok_body, grid=(2,),
        in_specs=in_specs,
        out_specs=pl.BlockSpec((512, 768), row),
        out_shape=jax.ShapeDtypeStruct((1024, 768), jnp.float32),
    )(agg, sel_feats, maskf, w_n1.T, b_n1.reshape(1, -1),
      w_n2.T, b_n2.reshape(1, -1))


def kernel(coords, features, batch_ids, times, tau, ln_g, ln_b,
           w_sp1, b_sp1, w_sp2, b_sp2,
           w_m1, b_m1, w_m2, b_m2, w_m3, b_m3, w_m4, b_m4,
           w_i1, b_i1, w_i2, b_i2, w_n1, b_n1, w_n2, b_n2):
    n = coords.shape[0]
    bsz, k_eff = 8, 128
    xyzt = jnp.concatenate([coords, times], axis=-1)
    point_feats, scores, counts = _run_mlp(
        xyzt, features, batch_ids, ln_g, ln_b,
        w_sp1, b_sp1, w_sp2, b_sp2, w_m1, b_m1, w_m2, b_m2,
        w_m3, b_m3, w_m4, b_m4, w_i1, b_i1, w_i2, b_i2)

    offsets = jnp.concatenate(
        [jnp.zeros((1,), counts.dtype), jnp.cumsum(counts)[:-1]])

    neg_inf = jnp.finfo(jnp.float32).min
    pos = jnp.arange(n, dtype=jnp.int32)
    pos_in_b = pos - offsets[batch_ids]
    S = jnp.full((bsz, n), neg_inf, jnp.float32).at[batch_ids, pos_in_b].set(scores)
    _, top_pos = jax.lax.top_k(S, k_eff)
    selpos = jnp.where(top_pos < counts[:, None], top_pos, 0)
    selpos = selpos.reshape(-1).astype(jnp.int32)

    pad = jnp.zeros((NPAD - n,), jnp.float32)
    xp = jnp.concatenate([coords[:, 0], pad])
    yp = jnp.concatenate([coords[:, 1], pad])
    zp = jnp.concatenate([coords[:, 2], pad])
    xyzt128 = jnp.concatenate([xyzt, jnp.zeros((n, 124), jnp.float32)], axis=1)
    agg, sel_feats, cent128 = _run_knn(
        xp, yp, zp, xyzt128, point_feats, selpos, counts, offsets)

    mask = jnp.arange(k_eff, dtype=jnp.int32)[None, :] < counts[:, None]
    maskf = mask.astype(jnp.float32).reshape(1024, 1)
    tokens = _run_tok(agg, sel_feats, maskf, w_n1, b_n1, w_n2, b_n2)
    tokens = tokens.reshape(bsz, k_eff, 768)
    centroids = (cent128[:, :4] * maskf).reshape(bsz, k_eff, 4)
    return tokens, centroids, mask


# speedup vs baseline: 23.7002x; 1.2989x over previous
import functools

import jax
import jax.numpy as jnp
from jax import lax
from jax.experimental import pallas as pl
from jax.experimental.pallas import tpu as pltpu
from jax.experimental.pallas import tpu_sc as plsc

N_POINTS = 32768
BLK = 512
W_WIN = 4096
NW = 8
NPAD = N_POINTS + W_WIN + 8
NSC = 32
F_INF = float("inf")


def _mlp_body(xyzt_ref, feats_ref, bids_ref,
              lng_ref, lnb_ref,
              wsp1_ref, bsp1_ref, wsp2_ref, bsp2_ref,
              wm1_ref, bm1_ref, wm2_ref, bm2_ref,
              wm3_ref, bm3_ref, wm4_ref, bm4_ref,
              wi1_ref, bi1_ref, wi2_ref, bi2_ref,
              pf_out_ref, sc_out_ref, cnt_out_ref):
    i = pl.program_id(0)
    xyzt = xyzt_ref[...]
    mu = jnp.mean(xyzt, axis=1, keepdims=True)
    var = jnp.mean((xyzt - mu) ** 2, axis=1, keepdims=True)
    xn = (xyzt - mu) / jnp.sqrt(var + 1e-5)
    xn = xn * lng_ref[...] + lnb_ref[...]
    f32 = jnp.float32
    h = jax.nn.relu(jnp.dot(xn, wsp1_ref[...], preferred_element_type=f32)
                    + bsp1_ref[...])
    sp = jax.nn.relu(jnp.dot(h, wsp2_ref[...], preferred_element_type=f32)
                     + bsp2_ref[...])
    pf = jnp.concatenate([feats_ref[...], sp], axis=1)
    pf = jax.nn.relu(jnp.dot(pf, wm1_ref[...], preferred_element_type=f32)
                     + bm1_ref[...])
    pf = jax.nn.relu(jnp.dot(pf, wm2_ref[...], preferred_element_type=f32)
                     + bm2_ref[...])
    pf = jax.nn.relu(jnp.dot(pf, wm3_ref[...], preferred_element_type=f32)
                     + bm3_ref[...])
    pf = jnp.dot(pf, wm4_ref[...], preferred_element_type=f32) + bm4_ref[...]
    pf_out_ref[...] = pf
    hi = jax.nn.relu(jnp.dot(pf, wi1_ref[...], preferred_element_type=f32)
                     + bi1_ref[...])
    sc_out_ref[...] = jnp.dot(hi, wi2_ref[...], preferred_element_type=f32) \
        + bi2_ref[...]
    bids = bids_ref[...]
    bins = jax.lax.broadcasted_iota(jnp.int32, (1, 8), 1)
    part = jnp.sum((bids == bins).astype(jnp.float32), axis=0, keepdims=True)
    prev = jnp.where(i == 0, jnp.zeros_like(part), cnt_out_ref[...])
    cnt_out_ref[...] = prev + part


def _run_mlp(xyzt, features, batch_ids,
             ln_g, ln_b, w_sp1, b_sp1, w_sp2, b_sp2,
             w_m1, b_m1, w_m2, b_m2, w_m3, b_m3, w_m4, b_m4,
             w_i1, b_i1, w_i2, b_i2):
    n = xyzt.shape[0]
    grid = n // BLK
    row = lambda i: (i, 0)
    fix = lambda i: (0, 0)

    def wspec(shape):
        return pl.BlockSpec(shape, fix)

    ws = [ln_g.reshape(1, 4), ln_b.reshape(1, 4),
          w_sp1.T, b_sp1.reshape(1, -1), w_sp2.T, b_sp2.reshape(1, -1),
          w_m1.T, b_m1.reshape(1, -1), w_m2.T, b_m2.reshape(1, -1),
          w_m3.T, b_m3.reshape(1, -1), w_m4.T, b_m4.reshape(1, -1),
          w_i1.T, b_i1.reshape(1, -1), w_i2.T, b_i2.reshape(1, -1)]
    in_specs = [pl.BlockSpec((BLK, 4), row),
                pl.BlockSpec((BLK, 256), row),
                pl.BlockSpec((BLK, 1), row)] + [wspec(w.shape) for w in ws]
    out_specs = [pl.BlockSpec((BLK, 768), row),
                 pl.BlockSpec((BLK, 1), row),
                 pl.BlockSpec((1, 8), fix)]
    out_shape = [jax.ShapeDtypeStruct((n, 768), jnp.float32),
                 jax.ShapeDtypeStruct((n, 1), jnp.float32),
                 jax.ShapeDtypeStruct((1, 8), jnp.float32)]
    pf, sc, cnt = pl.pallas_call(
        _mlp_body, grid=(grid,),
        in_specs=in_specs, out_specs=out_specs, out_shape=out_shape,
    )(xyzt, features, batch_ids.reshape(n, 1), *ws)
    return pf, sc[:, 0], cnt[0].astype(jnp.int32)


def _axc():
    return lax.axis_index("c")


def _axs():
    return lax.axis_index("s")


def _sload(ref, idx):
    return ref[pl.ds(idx, 16)][0]


def _knn_body(x_hbm, y_hbm, z_hbm, xyzt128_hbm, feats_hbm, sel_hbm, co_hbm,
              agg_hbm, selfeats_hbm, cent_hbm,
              xbuf, ybuf, zbuf, cobuf, selbuf, gidx, cidx, rows_v,
              centrows, aggrow, zrow, sem):
    wid = _axc() * 16 + _axs()
    seg = wid // 4
    q = wid % 4
    iota = jnp.arange(16, dtype=jnp.int32)

    pltpu.sync_copy(co_hbm, cobuf.at[pl.ds(0, 128)])
    cnt = _sload(cobuf, seg)
    off = _sload(cobuf, 8 + seg)
    sh = off % 8
    off_al = pl.multiple_of(off - sh, 8)
    lim = sh + cnt

    zf = jnp.zeros((16,), jnp.float32)

    def _zero(i, _):
        zrow[pl.ds(i * 16, 16)] = zf
        aggrow[pl.ds(i * 16, 16)] = zf
        return 0

    lax.fori_loop(0, 48, _zero, 0)

    for w in range(NW):

        @pl.when(w * W_WIN < lim)
        def _copy():
            wb = pl.multiple_of(off_al + w * W_WIN, 8)
            pltpu.sync_copy(x_hbm.at[pl.ds(wb, W_WIN)],
                            xbuf.at[pl.ds(w * W_WIN, W_WIN)])
            pltpu.sync_copy(y_hbm.at[pl.ds(wb, W_WIN)],
                            ybuf.at[pl.ds(w * W_WIN, W_WIN)])
            pltpu.sync_copy(z_hbm.at[pl.ds(wb, W_WIN)],
                            zbuf.at[pl.ds(w * W_WIN, W_WIN)])

    pltpu.sync_copy(sel_hbm, selbuf.at[pl.ds(0, 1024)])

    for h in range(2):
        selv = selbuf[pl.ds(wid * 32 + h * 16, 16)]
        cidx[...] = selv + off
        pltpu.async_copy(xyzt128_hbm.at[cidx], centrows, sem).wait()
        pltpu.sync_copy(centrows,
                        cent_hbm.at[pl.ds(wid * 32 + h * 16, 16), :])

    jlim = (lim + 15) // 16

    def _token(tloc, _):
        r = q * 32 + tloc
        gout = seg * 128 + r
        valid = r < cnt

        @pl.when(valid)
        def _do():
            pos = _sload(selbuf, wid * 32 + tloc)
            own_local = pos + sh
            own_g = off + pos
            tx = _sload(xbuf, own_local)
            ty = _sload(ybuf, own_local)
            tz = _sload(zbuf, own_local)

            def _scan(j, carry):
                bk, bv, th = carry
                base = j * 16
                xv = xbuf[pl.ds(base, 16)]
                yv = ybuf[pl.ds(base, 16)]
                zv = zbuf[pl.ds(base, 16)]
                loc = base + iota
                dx = xv - tx
                dy = yv - ty
                dz = zv - tz
                d = dx * dx + dy * dy + dz * dz
                ok = (loc >= sh) & (loc < lim)
                dm = jnp.where(ok, d, F_INF)
                hit = dm < th

                def _merge(args):
                    bk, bv, th = args
                    ck, cv = plsc.sort_key_val(dm, loc)
                    rk = jnp.flip(ck)
                    rv = jnp.flip(cv)
                    m = bk <= rk
                    nk = jnp.where(m, bk, rk)
                    nv = jnp.where(m, bv, rv)
                    sk, sv = plsc.sort_key_val(nk, nv)
                    nth = jnp.full((16,), jnp.max(sk))
                    return sk, sv, nth

                return lax.cond(jnp.any(hit), _merge, lambda a: a,
                                (bk, bv, th))

            bk0 = jnp.full((16,), F_INF, jnp.float32)
            bv0 = jnp.zeros((16,), jnp.int32)
            bk, bv, _th = lax.fori_loop(0, jlim, _scan, (bk0, bv0, bk0))

            nvalid = jnp.minimum(cnt, 16)
            gidx[...] = bv + (off - sh)
            pltpu.async_copy(feats_hbm.at[gidx], rows_v, sem).wait()

            def _zrow(j, _):
                for c in range(48):
                    rows_v[j, pl.ds(c * 16, 16)] = zf
                return 0

            lax.fori_loop(nvalid, 16, _zrow, 0)

            nf = nvalid.astype(jnp.float32)
            for c in range(48):
                acc = rows_v[0, pl.ds(c * 16, 16)]
                for j in range(1, 16):
                    acc = acc + rows_v[j, pl.ds(c * 16, 16)]
                aggrow[pl.ds(c * 16, 16)] = acc / nf
            pltpu.sync_copy(aggrow, agg_hbm.at[gout])
            pltpu.sync_copy(feats_hbm.at[own_g], selfeats_hbm.at[gout])

        @pl.when(jnp.logical_not(valid))
        def _skip():
            pltpu.sync_copy(zrow, agg_hbm.at[gout])
            pltpu.sync_copy(zrow, selfeats_hbm.at[gout])

        return 0

    lax.fori_loop(0, 32, _token, 0)


def _run_knn(xp, yp, zp, xyzt128, point_feats, selpos, counts, offsets):
    mesh = plsc.VectorSubcoreMesh(core_axis_name="c", subcore_axis_name="s",
                                  num_cores=2, num_subcores=16)
    co = jnp.concatenate(
        [counts, offsets, jnp.zeros((112,), jnp.int32)]).astype(jnp.int32)
    out_type = [jax.ShapeDtypeStruct((1024, 768), jnp.float32),
                jax.ShapeDtypeStruct((1024, 768), jnp.float32),
                jax.ShapeDtypeStruct((1024, 128), jnp.float32)]
    scratch = [pltpu.VMEM((NW * W_WIN + 16,), jnp.float32),
               pltpu.VMEM((NW * W_WIN + 16,), jnp.float32),
               pltpu.VMEM((NW * W_WIN + 16,), jnp.float32),
               pltpu.VMEM((144,), jnp.int32),
               pltpu.VMEM((1040,), jnp.int32),
               pltpu.VMEM((16,), jnp.int32),
               pltpu.VMEM((16,), jnp.int32),
               pltpu.VMEM((16, 768), jnp.float32),
               pltpu.VMEM((16, 128), jnp.float32),
               pltpu.VMEM((768,), jnp.float32),
               pltpu.VMEM((768,), jnp.float32),
               pltpu.SemaphoreType.DMA]
    fn = pl.kernel(_knn_body, out_type=out_type, mesh=mesh,
                   scratch_types=scratch,
                   compiler_params=pltpu.CompilerParams(
                       needs_layout_passes=False))
    return fn(xp, yp, zp, xyzt128, point_feats, selpos, co)


def _topk_body(s_hbm, co_hbm, sel_hbm, sbuf, cobuf, outbuf):
    wid = _axc() * 16 + _axs()
    iota = jnp.arange(16, dtype=jnp.int32)

    @pl.when(wid < 8)
    def _work():
        seg = wid
        pltpu.sync_copy(co_hbm, cobuf.at[pl.ds(0, 128)])
        cnt = _sload(cobuf, seg)
        off = _sload(cobuf, 8 + seg)
        sh = off % 8
        off_al = pl.multiple_of(off - sh, 8)
        lim = sh + cnt

        for w in range(NW):

            @pl.when(w * W_WIN < lim)
            def _copy():
                wb = pl.multiple_of(off_al + w * W_WIN, 8)
                pltpu.sync_copy(s_hbm.at[pl.ds(wb, W_WIN)],
                                sbuf.at[pl.ds(w * W_WIN, W_WIN)])

        jlim = (lim + 15) // 16
        inf_v = jnp.full((16,), F_INF, jnp.float32)
        zi = jnp.zeros((16,), jnp.int32)

        def _scan(j, st):
            base = j * 16
            sv = sbuf[pl.ds(base, 16)]
            key = 0.0 - sv
            loc = base + iota
            ok = (loc >= sh) & (loc < lim)
            km = jnp.where(ok, key, F_INF)
            th = st[16]
            hit = km < th

            def _merge(st):
                ks = list(st[:8])
                vs = list(st[8:16])
                ck, cv = plsc.sort_key_val(km, loc)
                for i in range(8):
                    rk = jnp.flip(ck)
                    rv = jnp.flip(cv)
                    m = ks[i] <= rk
                    nk = jnp.where(m, ks[i], rk)
                    nv = jnp.where(m, vs[i], rv)
                    hk = jnp.where(m, rk, ks[i])
                    hv = jnp.where(m, rv, vs[i])
                    ks[i], vs[i] = plsc.sort_key_val(nk, nv)
                    ck, cv = plsc.sort_key_val(hk, hv)
                nth = jnp.full((16,), jnp.max(ks[7]))
                return (*ks, *vs, nth)

            return lax.cond(jnp.any(hit), _merge, lambda s: s, st)

        init = tuple([inf_v] * 8 + [zi] * 8 + [inf_v])
        st = lax.fori_loop(0, jlim, _scan, init)
        for i in range(8):
            outbuf[pl.ds(i * 16, 16)] = jnp.maximum(st[8 + i] - sh, 0)
        pltpu.sync_copy(outbuf,
                        sel_hbm.at[pl.ds(pl.multiple_of(seg * 128, 128), 128)])


def _run_topk(sp, counts, offsets):
    mesh = plsc.VectorSubcoreMesh(core_axis_name="c", subcore_axis_name="s",
                                  num_cores=2, num_subcores=16)
    co = jnp.concatenate(
        [counts, offsets, jnp.zeros((112,), jnp.int32)]).astype(jnp.int32)
    scratch = [pltpu.VMEM((NW * W_WIN + 16,), jnp.float32),
               pltpu.VMEM((144,), jnp.int32),
               pltpu.VMEM((128,), jnp.int32)]
    fn = pl.kernel(_topk_body,
                   out_type=jax.ShapeDtypeStruct((1024,), jnp.int32),
                   mesh=mesh, scratch_types=scratch,
                   compiler_params=pltpu.CompilerParams(
                       needs_layout_passes=False))
    return fn(sp, co)


def _tok_body(agg_ref, sf_ref, m_ref, w1_ref, b1_ref, w2_ref, b2_ref,
              out_ref):
    f32 = jnp.float32
    h = jax.nn.relu(jnp.dot(agg_ref[...], w1_ref[...],
                            preferred_element_type=f32) + b1_ref[...])
    neigh = jnp.dot(h, w2_ref[...], preferred_element_type=f32) + b2_ref[...]
    out_ref[...] = (sf_ref[...] + neigh) * m_ref[...]


def _run_tok(agg, sel_feats, maskf, w_n1, b_n1, w_n2, b_n2):
    row = lambda i: (i, 0)
    fix = lambda i: (0, 0)
    in_specs = [pl.BlockSpec((512, 768), row),
                pl.BlockSpec((512, 768), row),
                pl.BlockSpec((512, 1), row),
                pl.BlockSpec((768, 768), fix),
                pl.BlockSpec((1, 768), fix),
                pl.BlockSpec((768, 768), fix),
                pl.BlockSpec((1, 768), fix)]
    return pl.pallas_call(
        _tok_body, grid=(2,),
        in_specs=in_specs,
        out_specs=pl.BlockSpec((512, 768), row),
        out_shape=jax.ShapeDtypeStruct((1024, 768), jnp.float32),
    )(agg, sel_feats, maskf, w_n1.T, b_n1.reshape(1, -1),
      w_n2.T, b_n2.reshape(1, -1))


def kernel(coords, features, batch_ids, times, tau, ln_g, ln_b,
           w_sp1, b_sp1, w_sp2, b_sp2,
           w_m1, b_m1, w_m2, b_m2, w_m3, b_m3, w_m4, b_m4,
           w_i1, b_i1, w_i2, b_i2, w_n1, b_n1, w_n2, b_n2):
    n = coords.shape[0]
    bsz, k_eff = 8, 128
    xyzt = jnp.concatenate([coords, times], axis=-1)
    point_feats, scores, counts = _run_mlp(
        xyzt, features, batch_ids, ln_g, ln_b,
        w_sp1, b_sp1, w_sp2, b_sp2, w_m1, b_m1, w_m2, b_m2,
        w_m3, b_m3, w_m4, b_m4, w_i1, b_i1, w_i2, b_i2)

    offsets = jnp.concatenate(
        [jnp.zeros((1,), counts.dtype), jnp.cumsum(counts)[:-1]])

    pad = jnp.zeros((NPAD - n,), jnp.float32)
    sp = jnp.concatenate([scores, pad])
    selpos = _run_topk(sp, counts, offsets)

    xp = jnp.concatenate([coords[:, 0], pad])
    yp = jnp.concatenate([coords[:, 1], pad])
    zp = jnp.concatenate([coords[:, 2], pad])
    xyzt128 = jnp.concatenate([xyzt, jnp.zeros((n, 124), jnp.float32)], axis=1)
    agg, sel_feats, cent128 = _run_knn(
        xp, yp, zp, xyzt128, point_feats, selpos, counts, offsets)

    mask = jnp.arange(k_eff, dtype=jnp.int32)[None, :] < counts[:, None]
    maskf = mask.astype(jnp.float32).reshape(1024, 1)
    tokens = _run_tok(agg, sel_feats, maskf, w_n1, b_n1, w_n2, b_n2)
    tokens = tokens.reshape(bsz, k_eff, 768)
    centroids = (cent128[:, :4] * maskf).reshape(bsz, k_eff, 4)
    return tokens, centroids, mask


# speedup vs baseline: 32.2692x; 1.3616x over previous
import functools

import jax
import jax.numpy as jnp
from jax import lax
from jax.experimental import pallas as pl
from jax.experimental.pallas import tpu as pltpu
from jax.experimental.pallas import tpu_sc as plsc

N_POINTS = 32768
BLK = 512
W_WIN = 4096
NW = 8
NPAD = N_POINTS + W_WIN + 8
NSC = 32
CAP = 1024
F_INF = float("inf")


def _mlp_body(xyzt_ref, feats_ref, bids_ref,
              lng_ref, lnb_ref,
              wsp1_ref, bsp1_ref, wsp2_ref, bsp2_ref,
              wm1_ref, bm1_ref, wm2_ref, bm2_ref,
              wm3_ref, bm3_ref, wm4_ref, bm4_ref,
              wi1_ref, bi1_ref, wi2_ref, bi2_ref,
              pf_out_ref, sc_out_ref, cnt_out_ref):
    i = pl.program_id(0)
    xyzt = xyzt_ref[...]
    mu = jnp.mean(xyzt, axis=1, keepdims=True)
    var = jnp.mean((xyzt - mu) ** 2, axis=1, keepdims=True)
    xn = (xyzt - mu) / jnp.sqrt(var + 1e-5)
    xn = xn * lng_ref[...] + lnb_ref[...]
    f32 = jnp.float32
    h = jax.nn.relu(jnp.dot(xn, wsp1_ref[...], preferred_element_type=f32)
                    + bsp1_ref[...])
    sp = jax.nn.relu(jnp.dot(h, wsp2_ref[...], preferred_element_type=f32)
                     + bsp2_ref[...])
    pf = jnp.concatenate([feats_ref[...], sp], axis=1)
    pf = jax.nn.relu(jnp.dot(pf, wm1_ref[...], preferred_element_type=f32)
                     + bm1_ref[...])
    pf = jax.nn.relu(jnp.dot(pf, wm2_ref[...], preferred_element_type=f32)
                     + bm2_ref[...])
    pf = jax.nn.relu(jnp.dot(pf, wm3_ref[...], preferred_element_type=f32)
                     + bm3_ref[...])
    pf = jnp.dot(pf, wm4_ref[...], preferred_element_type=f32) + bm4_ref[...]
    pf_out_ref[...] = pf
    hi = jax.nn.relu(jnp.dot(pf, wi1_ref[...], preferred_element_type=f32)
                     + bi1_ref[...])
    sc_out_ref[...] = jnp.dot(hi, wi2_ref[...], preferred_element_type=f32) \
        + bi2_ref[...]
    bids = bids_ref[...]
    bins = jax.lax.broadcasted_iota(jnp.int32, (1, 8), 1)
    part = jnp.sum((bids == bins).astype(jnp.float32), axis=0, keepdims=True)
    prev = jnp.where(i == 0, jnp.zeros_like(part), cnt_out_ref[...])
    cnt_out_ref[...] = prev + part


def _run_mlp(xyzt, features, batch_ids,
             ln_g, ln_b, w_sp1, b_sp1, w_sp2, b_sp2,
             w_m1, b_m1, w_m2, b_m2, w_m3, b_m3, w_m4, b_m4,
             w_i1, b_i1, w_i2, b_i2):
    n = xyzt.shape[0]
    grid = n // BLK
    row = lambda i: (i, 0)
    fix = lambda i: (0, 0)

    def wspec(shape):
        return pl.BlockSpec(shape, fix)

    ws = [ln_g.reshape(1, 4), ln_b.reshape(1, 4),
          w_sp1.T, b_sp1.reshape(1, -1), w_sp2.T, b_sp2.reshape(1, -1),
          w_m1.T, b_m1.reshape(1, -1), w_m2.T, b_m2.reshape(1, -1),
          w_m3.T, b_m3.reshape(1, -1), w_m4.T, b_m4.reshape(1, -1),
          w_i1.T, b_i1.reshape(1, -1), w_i2.T, b_i2.reshape(1, -1)]
    in_specs = [pl.BlockSpec((BLK, 4), row),
                pl.BlockSpec((BLK, 256), row),
                pl.BlockSpec((BLK, 1), row)] + [wspec(w.shape) for w in ws]
    out_specs = [pl.BlockSpec((BLK, 768), row),
                 pl.BlockSpec((BLK, 1), row),
                 pl.BlockSpec((1, 8), fix)]
    out_shape = [jax.ShapeDtypeStruct((n, 768), jnp.float32),
                 jax.ShapeDtypeStruct((n, 1), jnp.float32),
                 jax.ShapeDtypeStruct((1, 8), jnp.float32)]
    pf, sc, cnt = pl.pallas_call(
        _mlp_body, grid=(grid,),
        in_specs=in_specs, out_specs=out_specs, out_shape=out_shape,
    )(xyzt, features, batch_ids.reshape(n, 1), *ws)
    return pf, sc[:, 0], cnt[0].astype(jnp.int32)


def _axc():
    return lax.axis_index("c")


def _axs():
    return lax.axis_index("s")


def _sload(ref, idx):
    return ref[pl.ds(idx, 16)][0]


def _knn_body(x_hbm, y_hbm, z_hbm, xyzt128_hbm, feats_hbm, sel_hbm, co_hbm,
              agg_hbm, selfeats_hbm, cent_hbm,
              xbuf, ybuf, zbuf, cobuf, selbuf, gidx, cidx, rows_v,
              centrows, aggrow, zrow, dbuf, lbuf, resk, resv, sem):
    wid = _axc() * 16 + _axs()
    seg = wid // 4
    q = wid % 4
    iota = jnp.arange(16, dtype=jnp.int32)

    pltpu.sync_copy(co_hbm, cobuf.at[pl.ds(0, 128)])
    cnt = _sload(cobuf, seg)
    off = _sload(cobuf, 8 + seg)
    sh = off % 8
    off_al = pl.multiple_of(off - sh, 8)
    lim = sh + cnt

    zf = jnp.zeros((16,), jnp.float32)

    def _zero(i, _):
        zrow[pl.ds(i * 16, 16)] = zf
        aggrow[pl.ds(i * 16, 16)] = zf
        return 0

    lax.fori_loop(0, 48, _zero, 0)

    for w in range(NW):

        @pl.when(w * W_WIN < lim)
        def _copy():
            wb = pl.multiple_of(off_al + w * W_WIN, 8)
            pltpu.sync_copy(x_hbm.at[pl.ds(wb, W_WIN)],
                            xbuf.at[pl.ds(w * W_WIN, W_WIN)])
            pltpu.sync_copy(y_hbm.at[pl.ds(wb, W_WIN)],
                            ybuf.at[pl.ds(w * W_WIN, W_WIN)])
            pltpu.sync_copy(z_hbm.at[pl.ds(wb, W_WIN)],
                            zbuf.at[pl.ds(w * W_WIN, W_WIN)])

    inf_v = jnp.full((16,), F_INF, jnp.float32)
    for buf in (xbuf, ybuf, zbuf):
        hv = buf[pl.ds(0, 16)]
        buf[pl.ds(0, 16)] = jnp.where(iota < sh, inf_v, hv)
        buf[pl.ds(lim, 16)] = inf_v

    pltpu.sync_copy(sel_hbm, selbuf.at[pl.ds(0, 1024)])

    for h in range(2):
        selv = selbuf[pl.ds(wid * 32 + h * 16, 16)]
        cidx[...] = selv + off
        pltpu.async_copy(xyzt128_hbm.at[cidx], centrows, sem).wait()
        pltpu.sync_copy(centrows,
                        cent_hbm.at[pl.ds(wid * 32 + h * 16, 16), :])

    jlim = (lim + 15) // 16

    def _token(tloc, _):
        r = q * 32 + tloc
        gout = seg * 128 + r
        valid = r < cnt

        @pl.when(valid)
        def _do():
            pos = _sload(selbuf, wid * 32 + tloc)
            own_local = pos + sh
            own_g = off + pos
            tx = _sload(xbuf, own_local)
            ty = _sload(ybuf, own_local)
            tz = _sload(zbuf, own_local)

            bk0 = jnp.full((16,), F_INF, jnp.float32)
            bv0 = jnp.zeros((16,), jnp.int32)

            def _dist(base):
                xv = xbuf[pl.ds(base, 16)]
                yv = ybuf[pl.ds(base, 16)]
                zv = zbuf[pl.ds(base, 16)]
                dx = xv - tx
                dy = yv - ty
                dz = zv - tz
                return dx * dx + dy * dy + dz * dz

            def _p1(j, lmin):
                return jnp.minimum(lmin, _dist(j * 16))

            lmin = lax.fori_loop(0, jlim, _p1, bk0)
            t0v = jnp.full((16,), jnp.max(lmin))

            def _p2(j, ptr):
                d = _dist(j * 16)
                loc = j * 16 + iota
                msk = d <= t0v
                ptrc = jnp.minimum(ptr, CAP - 16)
                plsc.store_compressed(dbuf.at[pl.ds(ptrc, 16)], d, mask=msk)
                plsc.store_compressed(lbuf.at[pl.ds(ptrc, 16)], loc, mask=msk)
                c = plsc.all_reduce_population_count(msk)[0]
                return ptr + c

            ptr = lax.fori_loop(0, jlim, _p2, 0)
            okf = ptr <= CAP - 16

            @pl.when(okf)
            def _fast():
                dbuf[pl.ds(ptr, 16)] = bk0
                lbuf[pl.ds(ptr, 16)] = bv0

                def _p3(j, st):
                    bk, bv = st
                    ck, cv = plsc.sort_key_val(dbuf[pl.ds(j * 16, 16)],
                                               lbuf[pl.ds(j * 16, 16)])
                    rk = jnp.flip(ck)
                    rv = jnp.flip(cv)
                    m = bk <= rk
                    nk = jnp.where(m, bk, rk)
                    nv = jnp.where(m, bv, rv)
                    sk, sv = plsc.sort_key_val(nk, nv)
                    return (sk, sv)

                bk, bv = lax.fori_loop(0, (ptr + 15) // 16, _p3, (bk0, bv0))
                resk[...] = bk
                resv[...] = bv

            @pl.when(jnp.logical_not(okf))
            def _slowpath():
                def _scan(j, carry):
                    bk, bv, th = carry
                    base = j * 16
                    loc = base + iota
                    ok = (loc >= sh) & (loc < lim)
                    dm = jnp.where(ok, _dist(base), F_INF)
                    hit = dm < th

                    def _merge(args):
                        bk, bv, th = args
                        ck, cv = plsc.sort_key_val(dm, loc)
                        rk = jnp.flip(ck)
                        rv = jnp.flip(cv)
                        m = bk <= rk
                        nk = jnp.where(m, bk, rk)
                        nv = jnp.where(m, bv, rv)
                        sk, sv = plsc.sort_key_val(nk, nv)
                        nth = jnp.full((16,), jnp.max(sk))
                        return sk, sv, nth

                    return lax.cond(jnp.any(hit), _merge, lambda a: a,
                                    (bk, bv, th))

                bk, bv, _th = lax.fori_loop(0, jlim, _scan, (bk0, bv0, bk0))
                resk[...] = bk
                resv[...] = bv

            bv = resv[...]
            nvalid = jnp.minimum(cnt, 16)
            gidx[...] = jnp.minimum(bv + (off - sh), N_POINTS - 1)
            pltpu.async_copy(feats_hbm.at[gidx], rows_v, sem).wait()

            def _zrow(j, _):
                for c in range(48):
                    rows_v[j, pl.ds(c * 16, 16)] = zf
                return 0

            lax.fori_loop(nvalid, 16, _zrow, 0)

            nf = nvalid.astype(jnp.float32)
            for c in range(48):
                acc = rows_v[0, pl.ds(c * 16, 16)]
                for j in range(1, 16):
                    acc = acc + rows_v[j, pl.ds(c * 16, 16)]
                aggrow[pl.ds(c * 16, 16)] = acc / nf
            pltpu.sync_copy(aggrow, agg_hbm.at[gout])
            pltpu.sync_copy(feats_hbm.at[own_g], selfeats_hbm.at[gout])

        @pl.when(jnp.logical_not(valid))
        def _skip():
            pltpu.sync_copy(zrow, agg_hbm.at[gout])
            pltpu.sync_copy(zrow, selfeats_hbm.at[gout])

        return 0

    lax.fori_loop(0, 32, _token, 0)


def _run_knn(xp, yp, zp, xyzt128, point_feats, selpos, counts, offsets):
    mesh = plsc.VectorSubcoreMesh(core_axis_name="c", subcore_axis_name="s",
                                  num_cores=2, num_subcores=16)
    co = jnp.concatenate(
        [counts, offsets, jnp.zeros((112,), jnp.int32)]).astype(jnp.int32)
    out_type = [jax.ShapeDtypeStruct((1024, 768), jnp.float32),
                jax.ShapeDtypeStruct((1024, 768), jnp.float32),
                jax.ShapeDtypeStruct((1024, 128), jnp.float32)]
    scratch = [pltpu.VMEM((NW * W_WIN + 32,), jnp.float32),
               pltpu.VMEM((NW * W_WIN + 32,), jnp.float32),
               pltpu.VMEM((NW * W_WIN + 32,), jnp.float32),
               pltpu.VMEM((144,), jnp.int32),
               pltpu.VMEM((1040,), jnp.int32),
               pltpu.VMEM((16,), jnp.int32),
               pltpu.VMEM((16,), jnp.int32),
               pltpu.VMEM((16, 768), jnp.float32),
               pltpu.VMEM((16, 128), jnp.float32),
               pltpu.VMEM((768,), jnp.float32),
               pltpu.VMEM((768,), jnp.float32),
               pltpu.VMEM((CAP + 32,), jnp.float32),
               pltpu.VMEM((CAP + 32,), jnp.int32),
               pltpu.VMEM((16,), jnp.float32),
               pltpu.VMEM((16,), jnp.int32),
               pltpu.SemaphoreType.DMA]
    fn = pl.kernel(_knn_body, out_type=out_type, mesh=mesh,
                   scratch_types=scratch,
                   compiler_params=pltpu.CompilerParams(
                       needs_layout_passes=False))
    return fn(xp, yp, zp, xyzt128, point_feats, selpos, co)


def _topk_body(s_hbm, co_hbm, sel_hbm, sbuf, cobuf, outbuf):
    wid = _axc() * 16 + _axs()
    iota = jnp.arange(16, dtype=jnp.int32)

    @pl.when(wid < 8)
    def _work():
        seg = wid
        pltpu.sync_copy(co_hbm, cobuf.at[pl.ds(0, 128)])
        cnt = _sload(cobuf, seg)
        off = _sload(cobuf, 8 + seg)
        sh = off % 8
        off_al = pl.multiple_of(off - sh, 8)
        lim = sh + cnt

        for w in range(NW):

            @pl.when(w * W_WIN < lim)
            def _copy():
                wb = pl.multiple_of(off_al + w * W_WIN, 8)
                pltpu.sync_copy(s_hbm.at[pl.ds(wb, W_WIN)],
                                sbuf.at[pl.ds(w * W_WIN, W_WIN)])

        jlim = (lim + 15) // 16
        inf_v = jnp.full((16,), F_INF, jnp.float32)
        zi = jnp.zeros((16,), jnp.int32)

        def _scan(j, st):
            base = j * 16
            sv = sbuf[pl.ds(base, 16)]
            key = 0.0 - sv
            loc = base + iota
            ok = (loc >= sh) & (loc < lim)
            km = jnp.where(ok, key, F_INF)
            th = st[16]
            hit = km < th

            def _merge(st):
                ks = list(st[:8])
                vs = list(st[8:16])
                ck, cv = plsc.sort_key_val(km, loc)
                for i in range(8):
                    rk = jnp.flip(ck)
                    rv = jnp.flip(cv)
                    m = ks[i] <= rk
                    nk = jnp.where(m, ks[i], rk)
                    nv = jnp.where(m, vs[i], rv)
                    hk = jnp.where(m, rk, ks[i])
                    hv = jnp.where(m, rv, vs[i])
                    ks[i], vs[i] = plsc.sort_key_val(nk, nv)
                    ck, cv = plsc.sort_key_val(hk, hv)
                nth = jnp.full((16,), jnp.max(ks[7]))
                return (*ks, *vs, nth)

            return lax.cond(jnp.any(hit), _merge, lambda s: s, st)

        init = tuple([inf_v] * 8 + [zi] * 8 + [inf_v])
        st = lax.fori_loop(0, jlim, _scan, init)
        for i in range(8):
            outbuf[pl.ds(i * 16, 16)] = jnp.maximum(st[8 + i] - sh, 0)
        pltpu.sync_copy(outbuf,
                        sel_hbm.at[pl.ds(pl.multiple_of(seg * 128, 128), 128)])


def _run_topk(sp, counts, offsets):
    mesh = plsc.VectorSubcoreMesh(core_axis_name="c", subcore_axis_name="s",
                                  num_cores=2, num_subcores=16)
    co = jnp.concatenate(
        [counts, offsets, jnp.zeros((112,), jnp.int32)]).astype(jnp.int32)
    scratch = [pltpu.VMEM((NW * W_WIN + 16,), jnp.float32),
               pltpu.VMEM((144,), jnp.int32),
               pltpu.VMEM((128,), jnp.int32)]
    fn = pl.kernel(_topk_body,
                   out_type=jax.ShapeDtypeStruct((1024,), jnp.int32),
                   mesh=mesh, scratch_types=scratch,
                   compiler_params=pltpu.CompilerParams(
                       needs_layout_passes=False))
    return fn(sp, co)


def _tok_body(agg_ref, sf_ref, m_ref, w1_ref, b1_ref, w2_ref, b2_ref,
              out_ref):
    f32 = jnp.float32
    h = jax.nn.relu(jnp.dot(agg_ref[...], w1_ref[...],
                            preferred_element_type=f32) + b1_ref[...])
    neigh = jnp.dot(h, w2_ref[...], preferred_element_type=f32) + b2_ref[...]
    out_ref[...] = (sf_ref[...] + neigh) * m_ref[...]


def _run_tok(agg, sel_feats, maskf, w_n1, b_n1, w_n2, b_n2):
    row = lambda i: (i, 0)
    fix = lambda i: (0, 0)
    in_specs = [pl.BlockSpec((512, 768), row),
                pl.BlockSpec((512, 768), row),
                pl.BlockSpec((512, 1), row),
                pl.BlockSpec((768, 768), fix),
                pl.BlockSpec((1, 768), fix),
                pl.BlockSpec((768, 768), fix),
                pl.BlockSpec((1, 768), fix)]
    return pl.pallas_call(
        _tok_body, grid=(2,),
        in_specs=in_specs,
        out_specs=pl.BlockSpec((512, 768), row),
        out_shape=jax.ShapeDtypeStruct((1024, 768), jnp.float32),
    )(agg, sel_feats, maskf, w_n1.T, b_n1.reshape(1, -1),
      w_n2.T, b_n2.reshape(1, -1))


def kernel(coords, features, batch_ids, times, tau, ln_g, ln_b,
           w_sp1, b_sp1, w_sp2, b_sp2,
           w_m1, b_m1, w_m2, b_m2, w_m3, b_m3, w_m4, b_m4,
           w_i1, b_i1, w_i2, b_i2, w_n1, b_n1, w_n2, b_n2):
    n = coords.shape[0]
    bsz, k_eff = 8, 128
    xyzt = jnp.concatenate([coords, times], axis=-1)
    point_feats, scores, counts = _run_mlp(
        xyzt, features, batch_ids, ln_g, ln_b,
        w_sp1, b_sp1, w_sp2, b_sp2, w_m1, b_m1, w_m2, b_m2,
        w_m3, b_m3, w_m4, b_m4, w_i1, b_i1, w_i2, b_i2)

    offsets = jnp.concatenate(
        [jnp.zeros((1,), counts.dtype), jnp.cumsum(counts)[:-1]])

    pad = jnp.zeros((NPAD - n,), jnp.float32)
    sp = jnp.concatenate([scores, pad])
    selpos = _run_topk(sp, counts, offsets)

    xp = jnp.concatenate([coords[:, 0], pad])
    yp = jnp.concatenate([coords[:, 1], pad])
    zp = jnp.concatenate([coords[:, 2], pad])
    xyzt128 = jnp.concatenate([xyzt, jnp.zeros((n, 124), jnp.float32)], axis=1)
    agg, sel_feats, cent128 = _run_knn(
        xp, yp, zp, xyzt128, point_feats, selpos, counts, offsets)

    mask = jnp.arange(k_eff, dtype=jnp.int32)[None, :] < counts[:, None]
    maskf = mask.astype(jnp.float32).reshape(1024, 1)
    tokens = _run_tok(agg, sel_feats, maskf, w_n1, b_n1, w_n2, b_n2)
    tokens = tokens.reshape(bsz, k_eff, 768)
    centroids = (cent128[:, :4] * maskf).reshape(bsz, k_eff, 4)
    return tokens, centroids, mask


# speedup vs baseline: 33.2798x; 1.0313x over previous
import functools

import jax
import jax.numpy as jnp
from jax import lax
from jax.experimental import pallas as pl
from jax.experimental.pallas import tpu as pltpu
from jax.experimental.pallas import tpu_sc as plsc

N_POINTS = 32768
BLK = 512
W_WIN = 4096
NW = 8
NPAD = N_POINTS + W_WIN + 8
NSC = 32
CAP = 1024
F_INF = float("inf")


def _mlp_body(xyzt_ref, feats_ref, bids_ref,
              lng_ref, lnb_ref,
              wsp1_ref, bsp1_ref, wsp2_ref, bsp2_ref,
              wm1_ref, bm1_ref, wm2_ref, bm2_ref,
              wm3_ref, bm3_ref, wm4_ref, bm4_ref,
              wi1_ref, bi1_ref, wi2_ref, bi2_ref,
              pf_out_ref, sc_out_ref, cnt_out_ref):
    i = pl.program_id(0)
    xyzt = xyzt_ref[...]
    mu = jnp.mean(xyzt, axis=1, keepdims=True)
    var = jnp.mean((xyzt - mu) ** 2, axis=1, keepdims=True)
    xn = (xyzt - mu) / jnp.sqrt(var + 1e-5)
    xn = xn * lng_ref[...] + lnb_ref[...]
    f32 = jnp.float32
    h = jax.nn.relu(jnp.dot(xn, wsp1_ref[...], preferred_element_type=f32)
                    + bsp1_ref[...])
    sp = jax.nn.relu(jnp.dot(h, wsp2_ref[...], preferred_element_type=f32)
                     + bsp2_ref[...])
    pf = jnp.concatenate([feats_ref[...], sp], axis=1)
    pf = jax.nn.relu(jnp.dot(pf, wm1_ref[...], preferred_element_type=f32)
                     + bm1_ref[...])
    pf = jax.nn.relu(jnp.dot(pf, wm2_ref[...], preferred_element_type=f32)
                     + bm2_ref[...])
    pf = jax.nn.relu(jnp.dot(pf, wm3_ref[...], preferred_element_type=f32)
                     + bm3_ref[...])
    pf = jnp.dot(pf, wm4_ref[...], preferred_element_type=f32) + bm4_ref[...]
    pf_out_ref[...] = pf
    hi = jax.nn.relu(jnp.dot(pf, wi1_ref[...], preferred_element_type=f32)
                     + bi1_ref[...])
    sc_out_ref[...] = jnp.dot(hi, wi2_ref[...], preferred_element_type=f32) \
        + bi2_ref[...]
    bids = bids_ref[...]
    bins = jax.lax.broadcasted_iota(jnp.int32, (1, 8), 1)
    part = jnp.sum((bids == bins).astype(jnp.float32), axis=0, keepdims=True)
    prev = jnp.where(i == 0, jnp.zeros_like(part), cnt_out_ref[...])
    cnt_out_ref[...] = prev + part


def _run_mlp(xyzt, features, batch_ids,
             ln_g, ln_b, w_sp1, b_sp1, w_sp2, b_sp2,
             w_m1, b_m1, w_m2, b_m2, w_m3, b_m3, w_m4, b_m4,
             w_i1, b_i1, w_i2, b_i2):
    n = xyzt.shape[0]
    grid = n // BLK
    row = lambda i: (i, 0)
    fix = lambda i: (0, 0)

    def wspec(shape):
        return pl.BlockSpec(shape, fix)

    ws = [ln_g.reshape(1, 4), ln_b.reshape(1, 4),
          w_sp1.T, b_sp1.reshape(1, -1), w_sp2.T, b_sp2.reshape(1, -1),
          w_m1.T, b_m1.reshape(1, -1), w_m2.T, b_m2.reshape(1, -1),
          w_m3.T, b_m3.reshape(1, -1), w_m4.T, b_m4.reshape(1, -1),
          w_i1.T, b_i1.reshape(1, -1), w_i2.T, b_i2.reshape(1, -1)]
    in_specs = [pl.BlockSpec((BLK, 4), row),
                pl.BlockSpec((BLK, 256), row),
                pl.BlockSpec((BLK, 1), row)] + [wspec(w.shape) for w in ws]
    out_specs = [pl.BlockSpec((BLK, 768), row),
                 pl.BlockSpec((BLK, 1), row),
                 pl.BlockSpec((1, 8), fix)]
    out_shape = [jax.ShapeDtypeStruct((n, 768), jnp.float32),
                 jax.ShapeDtypeStruct((n, 1), jnp.float32),
                 jax.ShapeDtypeStruct((1, 8), jnp.float32)]
    pf, sc, cnt = pl.pallas_call(
        _mlp_body, grid=(grid,),
        in_specs=in_specs, out_specs=out_specs, out_shape=out_shape,
    )(xyzt, features, batch_ids.reshape(n, 1), *ws)
    return pf, sc[:, 0], cnt[0].astype(jnp.int32)


def _axc():
    return lax.axis_index("c")


def _axs():
    return lax.axis_index("s")


def _sload(ref, idx):
    return ref[pl.ds(idx, 16)][0]


def _knn_body(x_hbm, y_hbm, z_hbm, xyzt128_hbm, feats_hbm, sel_hbm, co_hbm,
              agg_hbm, selfeats_hbm, cent_hbm,
              xbuf, ybuf, zbuf, cobuf, selbuf, gidx, cidx, rows_v,
              centrows, aggrow, zrow, dbuf, lbuf, resk, resv, sem):
    wid = _axc() * 16 + _axs()
    seg = wid // 4
    q = wid % 4
    iota = jnp.arange(16, dtype=jnp.int32)

    pltpu.sync_copy(co_hbm, cobuf.at[pl.ds(0, 128)])
    cnt = _sload(cobuf, seg)
    off = _sload(cobuf, 8 + seg)
    sh = off % 8
    off_al = pl.multiple_of(off - sh, 8)
    lim = sh + cnt

    zf = jnp.zeros((16,), jnp.float32)

    def _zero(i, _):
        zrow[pl.ds(i * 16, 16)] = zf
        aggrow[pl.ds(i * 16, 16)] = zf
        return 0

    lax.fori_loop(0, 48, _zero, 0)

    for w in range(NW):

        @pl.when(w * W_WIN < lim)
        def _copy():
            wb = pl.multiple_of(off_al + w * W_WIN, 8)
            pltpu.sync_copy(x_hbm.at[pl.ds(wb, W_WIN)],
                            xbuf.at[pl.ds(w * W_WIN, W_WIN)])
            pltpu.sync_copy(y_hbm.at[pl.ds(wb, W_WIN)],
                            ybuf.at[pl.ds(w * W_WIN, W_WIN)])
            pltpu.sync_copy(z_hbm.at[pl.ds(wb, W_WIN)],
                            zbuf.at[pl.ds(w * W_WIN, W_WIN)])

    inf_v = jnp.full((16,), F_INF, jnp.float32)
    for buf in (xbuf, ybuf, zbuf):
        hv = buf[pl.ds(0, 16)]
        buf[pl.ds(0, 16)] = jnp.where(iota < sh, inf_v, hv)
        for t in range(4):
            buf[pl.ds(lim + t * 16, 16)] = inf_v

    pltpu.sync_copy(sel_hbm, selbuf.at[pl.ds(0, 1024)])

    for h in range(2):
        selv = selbuf[pl.ds(wid * 32 + h * 16, 16)]
        cidx[...] = selv + off
        pltpu.async_copy(xyzt128_hbm.at[cidx], centrows, sem).wait()
        pltpu.sync_copy(centrows,
                        cent_hbm.at[pl.ds(wid * 32 + h * 16, 16), :])

    jlim = (lim + 15) // 16

    def _token(tloc, _):
        r = q * 32 + tloc
        gout = seg * 128 + r
        valid = r < cnt

        @pl.when(valid)
        def _do():
            pos = _sload(selbuf, wid * 32 + tloc)
            own_local = pos + sh
            own_g = off + pos
            tx = _sload(xbuf, own_local)
            ty = _sload(ybuf, own_local)
            tz = _sload(zbuf, own_local)

            bk0 = jnp.full((16,), F_INF, jnp.float32)
            bv0 = jnp.zeros((16,), jnp.int32)

            def _dist(base):
                xv = xbuf[pl.ds(base, 16)]
                yv = ybuf[pl.ds(base, 16)]
                zv = zbuf[pl.ds(base, 16)]
                dx = xv - tx
                dy = yv - ty
                dz = zv - tz
                return dx * dx + dy * dy + dz * dz

            glim = (lim + 63) // 64

            def _p1(g, lmin):
                for t in range(4):
                    lmin = jnp.minimum(lmin, _dist(g * 64 + t * 16))
                return lmin

            lmin = lax.fori_loop(0, glim, _p1, bk0)
            t0v = jnp.full((16,), jnp.max(lmin))

            def _p2(g, ptr):
                for t in range(4):
                    base = g * 64 + t * 16
                    d = _dist(base)
                    loc = base + iota
                    msk = d <= t0v
                    ptrc = jnp.minimum(ptr, CAP - 16)
                    plsc.store_compressed(dbuf.at[pl.ds(ptrc, 16)], d,
                                          mask=msk)
                    plsc.store_compressed(lbuf.at[pl.ds(ptrc, 16)], loc,
                                          mask=msk)
                    ptr = ptr + plsc.all_reduce_population_count(msk)[0]
                return ptr

            ptr = lax.fori_loop(0, glim, _p2, 0)
            okf = ptr <= CAP - 16

            @pl.when(okf)
            def _fast():
                dbuf[pl.ds(ptr, 16)] = bk0
                lbuf[pl.ds(ptr, 16)] = bv0

                def _p3(j, st):
                    bk, bv = st
                    ck, cv = plsc.sort_key_val(dbuf[pl.ds(j * 16, 16)],
                                               lbuf[pl.ds(j * 16, 16)])
                    rk = jnp.flip(ck)
                    rv = jnp.flip(cv)
                    m = bk <= rk
                    nk = jnp.where(m, bk, rk)
                    nv = jnp.where(m, bv, rv)
                    sk, sv = plsc.sort_key_val(nk, nv)
                    return (sk, sv)

                bk, bv = lax.fori_loop(0, (ptr + 15) // 16, _p3, (bk0, bv0))
                resk[...] = bk
                resv[...] = bv

            @pl.when(jnp.logical_not(okf))
            def _slowpath():
                def _scan(j, carry):
                    bk, bv, th = carry
                    base = j * 16
                    loc = base + iota
                    ok = (loc >= sh) & (loc < lim)
                    dm = jnp.where(ok, _dist(base), F_INF)
                    hit = dm < th

                    def _merge(args):
                        bk, bv, th = args
                        ck, cv = plsc.sort_key_val(dm, loc)
                        rk = jnp.flip(ck)
                        rv = jnp.flip(cv)
                        m = bk <= rk
                        nk = jnp.where(m, bk, rk)
                        nv = jnp.where(m, bv, rv)
                        sk, sv = plsc.sort_key_val(nk, nv)
                        nth = jnp.full((16,), jnp.max(sk))
                        return sk, sv, nth

                    return lax.cond(jnp.any(hit), _merge, lambda a: a,
                                    (bk, bv, th))

                bk, bv, _th = lax.fori_loop(0, jlim, _scan, (bk0, bv0, bk0))
                resk[...] = bk
                resv[...] = bv

            bv = resv[...]
            nvalid = jnp.minimum(cnt, 16)
            gidx[...] = jnp.minimum(bv + (off - sh), N_POINTS - 1)
            pltpu.async_copy(feats_hbm.at[gidx], rows_v, sem).wait()

            def _zrow(j, _):
                for c in range(48):
                    rows_v[j, pl.ds(c * 16, 16)] = zf
                return 0

            lax.fori_loop(nvalid, 16, _zrow, 0)

            nf = nvalid.astype(jnp.float32)
            for c in range(48):
                acc = rows_v[0, pl.ds(c * 16, 16)]
                for j in range(1, 16):
                    acc = acc + rows_v[j, pl.ds(c * 16, 16)]
                aggrow[pl.ds(c * 16, 16)] = acc / nf
            pltpu.sync_copy(aggrow, agg_hbm.at[gout])
            pltpu.sync_copy(feats_hbm.at[own_g], selfeats_hbm.at[gout])

        @pl.when(jnp.logical_not(valid))
        def _skip():
            pltpu.sync_copy(zrow, agg_hbm.at[gout])
            pltpu.sync_copy(zrow, selfeats_hbm.at[gout])

        return 0

    lax.fori_loop(0, 32, _token, 0)


def _run_knn(xp, yp, zp, xyzt128, point_feats, selpos, counts, offsets):
    mesh = plsc.VectorSubcoreMesh(core_axis_name="c", subcore_axis_name="s",
                                  num_cores=2, num_subcores=16)
    co = jnp.concatenate(
        [counts, offsets, jnp.zeros((112,), jnp.int32)]).astype(jnp.int32)
    out_type = [jax.ShapeDtypeStruct((1024, 768), jnp.float32),
                jax.ShapeDtypeStruct((1024, 768), jnp.float32),
                jax.ShapeDtypeStruct((1024, 128), jnp.float32)]
    scratch = [pltpu.VMEM((NW * W_WIN + 80,), jnp.float32),
               pltpu.VMEM((NW * W_WIN + 80,), jnp.float32),
               pltpu.VMEM((NW * W_WIN + 80,), jnp.float32),
               pltpu.VMEM((144,), jnp.int32),
               pltpu.VMEM((1040,), jnp.int32),
               pltpu.VMEM((16,), jnp.int32),
               pltpu.VMEM((16,), jnp.int32),
               pltpu.VMEM((16, 768), jnp.float32),
               pltpu.VMEM((16, 128), jnp.float32),
               pltpu.VMEM((768,), jnp.float32),
               pltpu.VMEM((768,), jnp.float32),
               pltpu.VMEM((CAP + 32,), jnp.float32),
               pltpu.VMEM((CAP + 32,), jnp.int32),
               pltpu.VMEM((16,), jnp.float32),
               pltpu.VMEM((16,), jnp.int32),
               pltpu.SemaphoreType.DMA]
    fn = pl.kernel(_knn_body, out_type=out_type, mesh=mesh,
                   scratch_types=scratch,
                   compiler_params=pltpu.CompilerParams(
                       needs_layout_passes=False))
    return fn(xp, yp, zp, xyzt128, point_feats, selpos, co)


def _topk_body(s_hbm, co_hbm, sel_hbm, sbuf, cobuf, outbuf):
    wid = _axc() * 16 + _axs()
    iota = jnp.arange(16, dtype=jnp.int32)

    @pl.when(wid < 8)
    def _work():
        seg = wid
        pltpu.sync_copy(co_hbm, cobuf.at[pl.ds(0, 128)])
        cnt = _sload(cobuf, seg)
        off = _sload(cobuf, 8 + seg)
        sh = off % 8
        off_al = pl.multiple_of(off - sh, 8)
        lim = sh + cnt

        for w in range(NW):

            @pl.when(w * W_WIN < lim)
            def _copy():
                wb = pl.multiple_of(off_al + w * W_WIN, 8)
                pltpu.sync_copy(s_hbm.at[pl.ds(wb, W_WIN)],
                                sbuf.at[pl.ds(w * W_WIN, W_WIN)])

        jlim = (lim + 15) // 16
        inf_v = jnp.full((16,), F_INF, jnp.float32)
        zi = jnp.zeros((16,), jnp.int32)

        def _scan(j, st):
            base = j * 16
            sv = sbuf[pl.ds(base, 16)]
            key = 0.0 - sv
            loc = base + iota
            ok = (loc >= sh) & (loc < lim)
            km = jnp.where(ok, key, F_INF)
            th = st[16]
            hit = km < th

            def _merge(st):
                ks = list(st[:8])
                vs = list(st[8:16])
                ck, cv = plsc.sort_key_val(km, loc)
                for i in range(8):
                    rk = jnp.flip(ck)
                    rv = jnp.flip(cv)
                    m = ks[i] <= rk
                    nk = jnp.where(m, ks[i], rk)
                    nv = jnp.where(m, vs[i], rv)
                    hk = jnp.where(m, rk, ks[i])
                    hv = jnp.where(m, rv, vs[i])
                    ks[i], vs[i] = plsc.sort_key_val(nk, nv)
                    ck, cv = plsc.sort_key_val(hk, hv)
                nth = jnp.full((16,), jnp.max(ks[7]))
                return (*ks, *vs, nth)

            return lax.cond(jnp.any(hit), _merge, lambda s: s, st)

        init = tuple([inf_v] * 8 + [zi] * 8 + [inf_v])
        st = lax.fori_loop(0, jlim, _scan, init)
        for i in range(8):
            outbuf[pl.ds(i * 16, 16)] = jnp.maximum(st[8 + i] - sh, 0)
        pltpu.sync_copy(outbuf,
                        sel_hbm.at[pl.ds(pl.multiple_of(seg * 128, 128), 128)])


def _run_topk(sp, counts, offsets):
    mesh = plsc.VectorSubcoreMesh(core_axis_name="c", subcore_axis_name="s",
                                  num_cores=2, num_subcores=16)
    co = jnp.concatenate(
        [counts, offsets, jnp.zeros((112,), jnp.int32)]).astype(jnp.int32)
    scratch = [pltpu.VMEM((NW * W_WIN + 16,), jnp.float32),
               pltpu.VMEM((144,), jnp.int32),
               pltpu.VMEM((128,), jnp.int32)]
    fn = pl.kernel(_topk_body,
                   out_type=jax.ShapeDtypeStruct((1024,), jnp.int32),
                   mesh=mesh, scratch_types=scratch,
                   compiler_params=pltpu.CompilerParams(
                       needs_layout_passes=False))
    return fn(sp, co)


def _tok_body(agg_ref, sf_ref, m_ref, w1_ref, b1_ref, w2_ref, b2_ref,
              out_ref):
    f32 = jnp.float32
    h = jax.nn.relu(jnp.dot(agg_ref[...], w1_ref[...],
                            preferred_element_type=f32) + b1_ref[...])
    neigh = jnp.dot(h, w2_ref[...], preferred_element_type=f32) + b2_ref[...]
    out_ref[...] = (sf_ref[...] + neigh) * m_ref[...]


def _run_tok(agg, sel_feats, maskf, w_n1, b_n1, w_n2, b_n2):
    row = lambda i: (i, 0)
    fix = lambda i: (0, 0)
    in_specs = [pl.BlockSpec((512, 768), row),
                pl.BlockSpec((512, 768), row),
                pl.BlockSpec((512, 1), row),
                pl.BlockSpec((768, 768), fix),
                pl.BlockSpec((1, 768), fix),
                pl.BlockSpec((768, 768), fix),
                pl.BlockSpec((1, 768), fix)]
    return pl.pallas_call(
        _tok_body, grid=(2,),
        in_specs=in_specs,
        out_specs=pl.BlockSpec((512, 768), row),
        out_shape=jax.ShapeDtypeStruct((1024, 768), jnp.float32),
    )(agg, sel_feats, maskf, w_n1.T, b_n1.reshape(1, -1),
      w_n2.T, b_n2.reshape(1, -1))


def kernel(coords, features, batch_ids, times, tau, ln_g, ln_b,
           w_sp1, b_sp1, w_sp2, b_sp2,
           w_m1, b_m1, w_m2, b_m2, w_m3, b_m3, w_m4, b_m4,
           w_i1, b_i1, w_i2, b_i2, w_n1, b_n1, w_n2, b_n2):
    n = coords.shape[0]
    bsz, k_eff = 8, 128
    xyzt = jnp.concatenate([coords, times], axis=-1)
    point_feats, scores, counts = _run_mlp(
        xyzt, features, batch_ids, ln_g, ln_b,
        w_sp1, b_sp1, w_sp2, b_sp2, w_m1, b_m1, w_m2, b_m2,
        w_m3, b_m3, w_m4, b_m4, w_i1, b_i1, w_i2, b_i2)

    offsets = jnp.concatenate(
        [jnp.zeros((1,), counts.dtype), jnp.cumsum(counts)[:-1]])

    pad = jnp.zeros((NPAD - n,), jnp.float32)
    sp = jnp.concatenate([scores, pad])
    selpos = _run_topk(sp, counts, offsets)

    xp = jnp.concatenate([coords[:, 0], pad])
    yp = jnp.concatenate([coords[:, 1], pad])
    zp = jnp.concatenate([coords[:, 2], pad])
    xyzt128 = jnp.concatenate([xyzt, jnp.zeros((n, 124), jnp.float32)], axis=1)
    agg, sel_feats, cent128 = _run_knn(
        xp, yp, zp, xyzt128, point_feats, selpos, counts, offsets)

    mask = jnp.arange(k_eff, dtype=jnp.int32)[None, :] < counts[:, None]
    maskf = mask.astype(jnp.float32).reshape(1024, 1)
    tokens = _run_tok(agg, sel_feats, maskf, w_n1, b_n1, w_n2, b_n2)
    tokens = tokens.reshape(bsz, k_eff, 768)
    centroids = (cent128[:, :4] * maskf).reshape(bsz, k_eff, 4)
    return tokens, centroids, mask
